# baseline (speedup 1.0000x reference)
import functools

import jax
import jax.numpy as jnp
from jax import lax
from jax.experimental import pallas as pl
from jax.experimental.pallas import tpu as pltpu
from jax.experimental.pallas import tpu_sc as plsc

B = 16384
F = 26
E = 16
D0 = F * E
BF = B * F

_NC = 2
_NS = 16
_NW = _NC * _NS
_W_PER_W = BF // _NW
_WCH = 1664
_NCHUNK = _W_PER_W // _WCH
_VCH = _WCH * E

_BB = 1024


def _idx_body(i_ref, o_ref):
    idx = i_ref[...]
    e = lax.broadcasted_iota(jnp.int32, (1, 1, E), 2)
    o_ref[...] = (idx[:, :, None] * E + e).reshape(idx.shape[0], D0)


def _idx_call(inputs, interpret=False):
    return pl.pallas_call(
        _idx_body,
        grid=(B // _BB,),
        in_specs=[pl.BlockSpec((_BB, F), lambda i: (i, 0))],
        out_specs=pl.BlockSpec((_BB, D0), lambda i: (i, 0)),
        out_shape=jax.ShapeDtypeStruct((B, D0), jnp.int32),
        interpret=interpret,
    )(inputs)


def _sc_gather(v_flat, w_flat, eidx, idx):
    mesh = plsc.VectorSubcoreMesh(core_axis_name="c", subcore_axis_name="s")

    @functools.partial(
        pl.kernel,
        mesh=mesh,
        out_type=(
            jax.ShapeDtypeStruct((BF * E,), jnp.float32),
            jax.ShapeDtypeStruct((BF,), jnp.float32),
        ),
        scratch_types=[
            pltpu.VMEM((_VCH,), jnp.int32),
            pltpu.VMEM((_VCH,), jnp.float32),
            pltpu.VMEM((_WCH,), jnp.int32),
            pltpu.VMEM((_WCH,), jnp.float32),
        ],
    )
    def k(vt_hbm, wt_hbm, eidx_hbm, idx_hbm, ov_hbm, ow_hbm,
          eidx_v, vbuf, widx_v, wbuf):
        wid = lax.axis_index("s") * _NC + lax.axis_index("c")
        vbase = wid * _W_PER_W * E
        wbase = wid * _W_PER_W
        for c in range(_NCHUNK):
            vo = vbase + c * _VCH
            wo = wbase + c * _WCH
            pltpu.sync_copy(eidx_hbm.at[pl.ds(vo, _VCH)], eidx_v)
            pltpu.sync_copy(idx_hbm.at[pl.ds(wo, _WCH)], widx_v)
            pltpu.sync_copy(vt_hbm.at[eidx_v], vbuf)
            pltpu.sync_copy(wt_hbm.at[widx_v], wbuf)
            pltpu.sync_copy(vbuf, ov_hbm.at[pl.ds(vo, _VCH)])
            pltpu.sync_copy(wbuf, ow_hbm.at[pl.ds(wo, _WCH)])

    return k(v_flat, w_flat, eidx, idx)


def _tc_body(xg_ref, wg_ref, wc_ref, w2_ref, w3_ref, c1_ref, c2_ref, c3_ref,
             o_ref):
    x = xg_ref[...]
    xb = x.astype(jnp.bfloat16)
    acc = lax.dot_general(xb, wc_ref[...], (((1,), (0,)), ((), ())),
                          preferred_element_type=jnp.float32)
    h1 = jnp.maximum(acc[:, :256] + c1_ref[...], 0.0)
    s = acc[:, 256:272]
    sumsq = jnp.sum(x * x, axis=1, keepdims=True)
    fm = 0.5 * (jnp.sum(s * s, axis=1, keepdims=True) - sumsq)
    wsum = jnp.sum(wg_ref[...], axis=1, keepdims=True)
    h2 = jnp.maximum(
        lax.dot_general(h1.astype(jnp.bfloat16), w2_ref[...],
                        (((1,), (0,)), ((), ())),
                        preferred_element_type=jnp.float32) + c2_ref[...], 0.0)
    h3 = jnp.sum(h2 * w3_ref[...], axis=1, keepdims=True)
    o_ref[...] = jax.nn.sigmoid(fm + wsum + h3 + c3_ref[...])


def _tc_call(xg, wgr, wc, w2, w3, c1, c2, c3, interpret=False):
    const = lambda i: (0, 0)
    return pl.pallas_call(
        _tc_body,
        grid=(B // _BB,),
        in_specs=[
            pl.BlockSpec((_BB, D0), lambda i: (i, 0)),
            pl.BlockSpec((_BB, F), lambda i: (i, 0)),
            pl.BlockSpec((D0, 272), const),
            pl.BlockSpec((256, 128), const),
            pl.BlockSpec((1, 128), const),
            pl.BlockSpec((1, 256), const),
            pl.BlockSpec((1, 128), const),
            pl.BlockSpec((1, 1), const),
        ],
        out_specs=pl.BlockSpec((_BB, 1), lambda i: (i, 0)),
        out_shape=jax.ShapeDtypeStruct((B, 1), jnp.float32),
        interpret=interpret,
    )(xg, wgr, wc, w2, w3, c1, c2, c3)


def _fold_weights(W1, b1, W2, b2, W3, b3, w0,
                  bn1a_g, bn1a_b, bn1a_m, bn1a_v, bn1b_g, bn1b_b, bn1b_m,
                  bn1b_v, bn2a_g, bn2a_b, bn2a_m, bn2a_v, bn2b_g, bn2b_b,
                  bn2b_m, bn2b_v):
    def affine(g_a, b_a, m_a, v_a, g_b, b_b, m_b, v_b):
        sa = g_a * lax.rsqrt(v_a + 1e-5)
        ta = b_a - m_a * sa
        sb = g_b * lax.rsqrt(v_b + 1e-5)
        tb = b_b - m_b * sb
        return sa * sb, ta * sb + tb

    s1, t1 = affine(bn1a_g, bn1a_b, bn1a_m, bn1a_v,
                    bn1b_g, bn1b_b, bn1b_m, bn1b_v)
    s2, t2 = affine(bn2a_g, bn2a_b, bn2a_m, bn2a_v,
                    bn2b_g, bn2b_b, bn2b_m, bn2b_v)
    w1f = (W1 * s1[:, None]).T
    ident = jnp.tile(jnp.eye(E, dtype=jnp.float32), (F, 1))
    wc = jnp.concatenate([w1f, ident], axis=1).astype(jnp.bfloat16)
    c1 = (b1 * s1 + t1)[None, :]
    w2f = ((W2 * s2[:, None]).T).astype(jnp.bfloat16)
    c2 = (b2 * s2 + t2)[None, :]
    c3 = (b3 + w0).reshape(1, 1)
    return wc, w2f, W3, c1, c2, c3


def kernel(inputs, w_table, v_table, w0, W1, b1, W2, b2, W3, b3,
           bn1a_g, bn1a_b, bn1a_m, bn1a_v, bn1b_g, bn1b_b, bn1b_m, bn1b_v,
           bn2a_g, bn2a_b, bn2a_m, bn2a_v, bn2b_g, bn2b_b, bn2b_m, bn2b_v):
    eidx = _idx_call(inputs)
    vgf, wgf = _sc_gather(v_table.reshape(-1), w_table.reshape(-1),
                          eidx.reshape(-1), inputs.reshape(-1))
    xg = vgf.reshape(B, D0)
    wgr = wgf.reshape(B, F)
    wc, w2f, w3, c1, c2, c3 = _fold_weights(
        W1, b1, W2, b2, W3, b3, w0,
        bn1a_g, bn1a_b, bn1a_m, bn1a_v, bn1b_g, bn1b_b, bn1b_m, bn1b_v,
        bn2a_g, bn2a_b, bn2a_m, bn2a_v, bn2b_g, bn2b_b, bn2b_m, bn2b_v)
    return _tc_call(xg, wgr, wc, w2f, w3, c1, c2, c3)


# speedup vs baseline: 1.1026x; 1.1026x over previous
import dataclasses
import functools

import jax
import jax.numpy as jnp
from jax import lax
from jax.experimental import pallas as pl
from jax.experimental.pallas import tpu as pltpu
from jax.experimental.pallas import tpu_sc as plsc

B = 16384
F = 26
E = 16
D0 = F * E
BF = B * F
VOCAB = 1000000
VROWS = VOCAB * E // 128

_NC = 2
_NS = 16
_NW = _NC * _NS
_PER_W = BF // _NW
_CH = 512
_NCHUNK = _PER_W // _CH

_BB = 1024


def _sc_gather(v128, w_flat, idxf):
    mesh = plsc.VectorSubcoreMesh(core_axis_name="c", subcore_axis_name="s")
    cp = pltpu.CompilerParams()
    if "needs_layout_passes" in pltpu.CompilerParams.__dataclass_fields__:
        cp = dataclasses.replace(cp, needs_layout_passes=False)

    @functools.partial(
        pl.kernel,
        mesh=mesh,
        compiler_params=cp,
        out_type=(
            jax.ShapeDtypeStruct((BF * E,), jnp.float32),
            jax.ShapeDtypeStruct((BF,), jnp.float32),
        ),
        scratch_types=[
            pltpu.VMEM((_CH,), jnp.int32),
            pltpu.VMEM((_CH,), jnp.int32),
            pltpu.VMEM((_CH,), jnp.int32),
            pltpu.VMEM((_CH, 128), jnp.float32),
            pltpu.VMEM((_CH * E,), jnp.float32),
            pltpu.VMEM((_CH,), jnp.float32),
        ],
    )
    def k(vt_hbm, wt_hbm, idx_hbm, ov_hbm, ow_hbm,
          idx_v, ridx_v, off_v, rows_v, vbuf, wbuf):
        wid = lax.axis_index("s") * _NC + lax.axis_index("c")
        base = wid * _PER_W
        iota16 = lax.iota(jnp.int32, 16)
        for c in range(_NCHUNK):
            o = base + c * _CH
            pltpu.sync_copy(idx_hbm.at[pl.ds(o, _CH)], idx_v)

            @pl.loop(0, _CH, step=16)
            def _(j):
                reg = idx_v[pl.ds(j, 16)]
                ridx_v[pl.ds(j, 16)] = lax.shift_right_logical(reg, 3)
                off_v[pl.ds(j, 16)] = lax.shift_left(
                    lax.bitwise_and(reg, 7), 4)

            pltpu.sync_copy(vt_hbm.at[ridx_v], rows_v)
            pltpu.sync_copy(wt_hbm.at[idx_v], wbuf)

            @pl.loop(0, _CH)
            def _(t):
                t16 = jnp.full((16,), t, jnp.int32)
                off = plsc.load_gather(off_v, [t16])
                vals = plsc.load_gather(rows_v, [t16, off + iota16])
                vbuf[pl.ds(t * 16, 16)] = vals

            pltpu.sync_copy(vbuf, ov_hbm.at[pl.ds(o * E, _CH * E)])
            pltpu.sync_copy(wbuf, ow_hbm.at[pl.ds(o, _CH)])

    return k(v128, w_flat, idxf)


def _tc_body(xg_ref, wg_ref, wc_ref, w2_ref, w3_ref, c1_ref, c2_ref, c3_ref,
             o_ref):
    x = xg_ref[...]
    xb = x.astype(jnp.bfloat16)
    acc = lax.dot_general(xb, wc_ref[...], (((1,), (0,)), ((), ())),
                          preferred_element_type=jnp.float32)
    h1 = jnp.maximum(acc[:, :256] + c1_ref[...], 0.0)
    s = acc[:, 256:272]
    sumsq = jnp.sum(x * x, axis=1, keepdims=True)
    fm = 0.5 * (jnp.sum(s * s, axis=1, keepdims=True) - sumsq)
    wsum = jnp.sum(wg_ref[...], axis=1, keepdims=True)
    h2 = jnp.maximum(
        lax.dot_general(h1.astype(jnp.bfloat16), w2_ref[...],
                        (((1,), (0,)), ((), ())),
                        preferred_element_type=jnp.float32) + c2_ref[...], 0.0)
    h3 = jnp.sum(h2 * w3_ref[...], axis=1, keepdims=True)
    o_ref[...] = jax.nn.sigmoid(fm + wsum + h3 + c3_ref[...])


def _tc_call(xg, wgr, wc, w2, w3, c1, c2, c3, interpret=False):
    const = lambda i: (0, 0)
    return pl.pallas_call(
        _tc_body,
        grid=(B // _BB,),
        in_specs=[
            pl.BlockSpec((_BB, D0), lambda i: (i, 0)),
            pl.BlockSpec((_BB, F), lambda i: (i, 0)),
            pl.BlockSpec((D0, 272), const),
            pl.BlockSpec((256, 128), const),
            pl.BlockSpec((1, 128), const),
            pl.BlockSpec((1, 256), const),
            pl.BlockSpec((1, 128), const),
            pl.BlockSpec((1, 1), const),
        ],
        out_specs=pl.BlockSpec((_BB, 1), lambda i: (i, 0)),
        out_shape=jax.ShapeDtypeStruct((B, 1), jnp.float32),
        interpret=interpret,
    )(xg, wgr, wc, w2, w3, c1, c2, c3)


def _fold_weights(W1, b1, W2, b2, W3, b3, w0,
                  bn1a_g, bn1a_b, bn1a_m, bn1a_v, bn1b_g, bn1b_b, bn1b_m,
                  bn1b_v, bn2a_g, bn2a_b, bn2a_m, bn2a_v, bn2b_g, bn2b_b,
                  bn2b_m, bn2b_v):
    def affine(g_a, b_a, m_a, v_a, g_b, b_b, m_b, v_b):
        sa = g_a * lax.rsqrt(v_a + 1e-5)
        ta = b_a - m_a * sa
        sb = g_b * lax.rsqrt(v_b + 1e-5)
        tb = b_b - m_b * sb
        return sa * sb, ta * sb + tb

    s1, t1 = affine(bn1a_g, bn1a_b, bn1a_m, bn1a_v,
                    bn1b_g, bn1b_b, bn1b_m, bn1b_v)
    s2, t2 = affine(bn2a_g, bn2a_b, bn2a_m, bn2a_v,
                    bn2b_g, bn2b_b, bn2b_m, bn2b_v)
    w1f = (W1 * s1[:, None]).T
    ident = jnp.tile(jnp.eye(E, dtype=jnp.float32), (F, 1))
    wc = jnp.concatenate([w1f, ident], axis=1).astype(jnp.bfloat16)
    c1 = (b1 * s1 + t1)[None, :]
    w2f = ((W2 * s2[:, None]).T).astype(jnp.bfloat16)
    c2 = (b2 * s2 + t2)[None, :]
    c3 = (b3 + w0).reshape(1, 1)
    return wc, w2f, W3, c1, c2, c3


def kernel(inputs, w_table, v_table, w0, W1, b1, W2, b2, W3, b3,
           bn1a_g, bn1a_b, bn1a_m, bn1a_v, bn1b_g, bn1b_b, bn1b_m, bn1b_v,
           bn2a_g, bn2a_b, bn2a_m, bn2a_v, bn2b_g, bn2b_b, bn2b_m, bn2b_v):
    v128 = lax.optimization_barrier(v_table.reshape(VROWS, 128))
    vgf, wgf = _sc_gather(v128, w_table.reshape(-1), inputs.reshape(-1))
    xg = vgf.reshape(B, D0)
    wgr = wgf.reshape(B, F)
    wc, w2f, w3, c1, c2, c3 = _fold_weights(
        W1, b1, W2, b2, W3, b3, w0,
        bn1a_g, bn1a_b, bn1a_m, bn1a_v, bn1b_g, bn1b_b, bn1b_m, bn1b_v,
        bn2a_g, bn2a_b, bn2a_m, bn2a_v, bn2b_g, bn2b_b, bn2b_m, bn2b_v)
    return _tc_call(xg, wgr, wc, w2f, w3, c1, c2, c3)


# speedup vs baseline: 1.1220x; 1.0176x over previous
import dataclasses
import functools

import jax
import jax.numpy as jnp
from jax import lax
from jax.experimental import pallas as pl
from jax.experimental.pallas import tpu as pltpu
from jax.experimental.pallas import tpu_sc as plsc

B = 16384
F = 26
E = 16
D0 = F * E
BF = B * F
VOCAB = 1000000
VROWS = VOCAB * E // 128

_NC = 2
_NS = 16
_NW = _NC * _NS
_PER_W = BF // _NW
_CH = 512
_NCHUNK = _PER_W // _CH

_BB = 1024


_SCH = 128
_S_PER_W = B // _NW
_NSCH = _S_PER_W // _SCH
_LCH = _SCH * F
_ICH = 416
_NICH = _LCH // _ICH


def _sc_gather(v128, w_flat, idx_t):
    mesh = plsc.VectorSubcoreMesh(core_axis_name="c", subcore_axis_name="s")
    cp = pltpu.CompilerParams()
    if "needs_layout_passes" in pltpu.CompilerParams.__dataclass_fields__:
        cp = dataclasses.replace(cp, needs_layout_passes=False)

    @functools.partial(
        pl.kernel,
        mesh=mesh,
        compiler_params=cp,
        out_type=(
            jax.ShapeDtypeStruct((BF * E,), jnp.float32),
            jax.ShapeDtypeStruct((BF,), jnp.float32),
        ),
        scratch_types=[
            pltpu.VMEM((F, _SCH), jnp.int32),
            pltpu.VMEM((_LCH,), jnp.int32),
            pltpu.VMEM((_LCH,), jnp.int32),
            pltpu.VMEM((_LCH,), jnp.int32),
            pltpu.VMEM((_ICH, 128), jnp.float32),
            pltpu.VMEM((_ICH * E,), jnp.float32),
            pltpu.VMEM((_LCH,), jnp.float32),
        ],
    )
    def k(vt_hbm, wt_hbm, idxt_hbm, ov_hbm, ow_hbm,
          idx2d, idx_v, ridx_v, off_v, rows_v, vbuf, wbuf):
        wid = lax.axis_index("s") * _NC + lax.axis_index("c")
        sbase = wid * _S_PER_W
        iota16 = lax.iota(jnp.int32, 16)
        fidx_lo = iota16
        fidx_hi = iota16 + (F - 16)
        for sc in range(_NSCH):
            b0 = sbase + sc * _SCH
            o = b0 * F
            pltpu.sync_copy(idxt_hbm.at[:, pl.ds(b0, _SCH)], idx2d)

            @pl.loop(0, _SCH)
            def _(b):
                b16 = jnp.full((16,), b, jnp.int32)
                lo = plsc.load_gather(idx2d, [fidx_lo, b16])
                hi = plsc.load_gather(idx2d, [fidx_hi, b16])
                idx_v[pl.ds(b * F, 16)] = lo
                idx_v[pl.ds(b * F + (F - 16), 16)] = hi

            @pl.loop(0, _LCH, step=16)
            def _(j):
                reg = idx_v[pl.ds(j, 16)]
                ridx_v[pl.ds(j, 16)] = lax.shift_right_logical(reg, 3)
                off_v[pl.ds(j, 16)] = lax.shift_left(
                    lax.bitwise_and(reg, 7), 4)

            pltpu.sync_copy(wt_hbm.at[idx_v], wbuf)
            pltpu.sync_copy(wbuf, ow_hbm.at[pl.ds(o, _LCH)])

            for ic in range(_NICH):
                go = ic * _ICH
                pltpu.sync_copy(vt_hbm.at[ridx_v.at[pl.ds(go, _ICH)]],
                                rows_v)

                @pl.loop(0, _ICH)
                def _(t):
                    t16 = jnp.full((16,), t, jnp.int32)
                    off = plsc.load_gather(off_v, [t16 + go])
                    vals = plsc.load_gather(rows_v, [t16, off + iota16])
                    vbuf[pl.ds(t * 16, 16)] = vals

                pltpu.sync_copy(
                    vbuf, ov_hbm.at[pl.ds((o + go) * E, _ICH * E)])

    return k(v128, w_flat, idx_t)


def _tc_body(xg_ref, wg_ref, wc_ref, w2_ref, w3_ref, c1_ref, c2_ref, c3_ref,
             o_ref):
    x = xg_ref[...]
    xb = x.astype(jnp.bfloat16)
    acc = lax.dot_general(xb, wc_ref[...], (((1,), (0,)), ((), ())),
                          preferred_element_type=jnp.float32)
    h1 = jnp.maximum(acc[:, :256] + c1_ref[...], 0.0)
    s = acc[:, 256:272]
    sumsq = jnp.sum(x * x, axis=1, keepdims=True)
    fm = 0.5 * (jnp.sum(s * s, axis=1, keepdims=True) - sumsq)
    wsum = jnp.sum(wg_ref[...], axis=1, keepdims=True)
    h2 = jnp.maximum(
        lax.dot_general(h1.astype(jnp.bfloat16), w2_ref[...],
                        (((1,), (0,)), ((), ())),
                        preferred_element_type=jnp.float32) + c2_ref[...], 0.0)
    h3 = jnp.sum(h2 * w3_ref[...], axis=1, keepdims=True)
    o_ref[...] = jax.nn.sigmoid(fm + wsum + h3 + c3_ref[...])


def _tc_call(xg, wgr, wc, w2, w3, c1, c2, c3, interpret=False):
    const = lambda i: (0, 0)
    return pl.pallas_call(
        _tc_body,
        grid=(B // _BB,),
        in_specs=[
            pl.BlockSpec((_BB, D0), lambda i: (i, 0)),
            pl.BlockSpec((_BB, F), lambda i: (i, 0)),
            pl.BlockSpec((D0, 272), const),
            pl.BlockSpec((256, 128), const),
            pl.BlockSpec((1, 128), const),
            pl.BlockSpec((1, 256), const),
            pl.BlockSpec((1, 128), const),
            pl.BlockSpec((1, 1), const),
        ],
        out_specs=pl.BlockSpec((_BB, 1), lambda i: (i, 0)),
        out_shape=jax.ShapeDtypeStruct((B, 1), jnp.float32),
        interpret=interpret,
    )(xg, wgr, wc, w2, w3, c1, c2, c3)


def _fold_weights(W1, b1, W2, b2, W3, b3, w0,
                  bn1a_g, bn1a_b, bn1a_m, bn1a_v, bn1b_g, bn1b_b, bn1b_m,
                  bn1b_v, bn2a_g, bn2a_b, bn2a_m, bn2a_v, bn2b_g, bn2b_b,
                  bn2b_m, bn2b_v):
    def affine(g_a, b_a, m_a, v_a, g_b, b_b, m_b, v_b):
        sa = g_a * lax.rsqrt(v_a + 1e-5)
        ta = b_a - m_a * sa
        sb = g_b * lax.rsqrt(v_b + 1e-5)
        tb = b_b - m_b * sb
        return sa * sb, ta * sb + tb

    s1, t1 = affine(bn1a_g, bn1a_b, bn1a_m, bn1a_v,
                    bn1b_g, bn1b_b, bn1b_m, bn1b_v)
    s2, t2 = affine(bn2a_g, bn2a_b, bn2a_m, bn2a_v,
                    bn2b_g, bn2b_b, bn2b_m, bn2b_v)
    w1f = (W1 * s1[:, None]).T
    ident = jnp.tile(jnp.eye(E, dtype=jnp.float32), (F, 1))
    wc = jnp.concatenate([w1f, ident], axis=1).astype(jnp.bfloat16)
    c1 = (b1 * s1 + t1)[None, :]
    w2f = ((W2 * s2[:, None]).T).astype(jnp.bfloat16)
    c2 = (b2 * s2 + t2)[None, :]
    c3 = (b3 + w0).reshape(1, 1)
    return wc, w2f, W3, c1, c2, c3


def kernel(inputs, w_table, v_table, w0, W1, b1, W2, b2, W3, b3,
           bn1a_g, bn1a_b, bn1a_m, bn1a_v, bn1b_g, bn1b_b, bn1b_m, bn1b_v,
           bn2a_g, bn2a_b, bn2a_m, bn2a_v, bn2b_g, bn2b_b, bn2b_m, bn2b_v):
    v128 = lax.optimization_barrier(v_table.reshape(VROWS, 128))
    vgf, wgf = _sc_gather(v128, w_table.reshape(-1), inputs.T)
    xg = vgf.reshape(B, D0)
    wgr = wgf.reshape(B, F)
    wc, w2f, w3, c1, c2, c3 = _fold_weights(
        W1, b1, W2, b2, W3, b3, w0,
        bn1a_g, bn1a_b, bn1a_m, bn1a_v, bn1b_g, bn1b_b, bn1b_m, bn1b_v,
        bn2a_g, bn2a_b, bn2a_m, bn2a_v, bn2b_g, bn2b_b, bn2b_m, bn2b_v)
    return _tc_call(xg, wgr, wc, w2f, w3, c1, c2, c3)


# speedup vs baseline: 1.1267x; 1.0042x over previous
import dataclasses
import functools

import jax
import jax.numpy as jnp
from jax import lax
from jax.experimental import pallas as pl
from jax.experimental.pallas import tpu as pltpu
from jax.experimental.pallas import tpu_sc as plsc

B = 16384
F = 26
E = 16
D0 = F * E
BF = B * F
VOCAB = 1000000
VROWS = VOCAB * E // 128

_NC = 2
_NS = 16
_NW = _NC * _NS
_PER_W = BF // _NW
_CH = 512
_NCHUNK = _PER_W // _CH

_BB = 1024


_SCH = 128
_S_PER_W = B // _NW
_NSCH = _S_PER_W // _SCH
_LCH = _SCH * F
_ICH = 416
_NICH = _LCH // _ICH


def _sc_gather(v128, w_flat, idx_t):
    mesh = plsc.VectorSubcoreMesh(core_axis_name="c", subcore_axis_name="s")
    cp = pltpu.CompilerParams()
    if "needs_layout_passes" in pltpu.CompilerParams.__dataclass_fields__:
        cp = dataclasses.replace(cp, needs_layout_passes=False)

    @functools.partial(
        pl.kernel,
        mesh=mesh,
        compiler_params=cp,
        out_type=(
            jax.ShapeDtypeStruct((BF * E,), jnp.float32),
            jax.ShapeDtypeStruct((BF,), jnp.float32),
        ),
        scratch_types=[
            pltpu.VMEM((_SCH, F), jnp.int32),
            pltpu.VMEM((_LCH,), jnp.int32),
            pltpu.VMEM((_LCH,), jnp.int32),
            pltpu.VMEM((_LCH,), jnp.int32),
            pltpu.VMEM((_ICH, 128), jnp.float32),
            pltpu.VMEM((_ICH * E,), jnp.float32),
            pltpu.VMEM((_LCH,), jnp.float32),
        ],
    )
    def k(vt_hbm, wt_hbm, idxt_hbm, ov_hbm, ow_hbm,
          idx2d, idx_v, ridx_v, off_v, rows_v, vbuf, wbuf):
        wid = lax.axis_index("s") * _NC + lax.axis_index("c")
        sbase = wid * _S_PER_W
        iota16 = lax.iota(jnp.int32, 16)
        fidx_lo = iota16
        fidx_hi = iota16 + (F - 16)
        for sc in range(_NSCH):
            b0 = sbase + sc * _SCH
            o = b0 * F
            pltpu.sync_copy(idxt_hbm.at[pl.ds(b0, _SCH)], idx2d)

            @pl.loop(0, _SCH)
            def _(b):
                b16 = jnp.full((16,), b, jnp.int32)
                lo = plsc.load_gather(idx2d, [b16, fidx_lo])
                hi = plsc.load_gather(idx2d, [b16, fidx_hi])
                idx_v[pl.ds(b * F, 16)] = lo
                idx_v[pl.ds(b * F + (F - 16), 16)] = hi

            @pl.loop(0, _LCH, step=16)
            def _(j):
                reg = idx_v[pl.ds(j, 16)]
                ridx_v[pl.ds(j, 16)] = lax.shift_right_logical(reg, 3)
                off_v[pl.ds(j, 16)] = lax.shift_left(
                    lax.bitwise_and(reg, 7), 4)

            pltpu.sync_copy(wt_hbm.at[idx_v], wbuf)
            pltpu.sync_copy(wbuf, ow_hbm.at[pl.ds(o, _LCH)])

            for ic in range(_NICH):
                go = ic * _ICH
                pltpu.sync_copy(vt_hbm.at[ridx_v.at[pl.ds(go, _ICH)]],
                                rows_v)

                @pl.loop(0, _ICH)
                def _(t):
                    t16 = jnp.full((16,), t, jnp.int32)
                    off = plsc.load_gather(off_v, [t16 + go])
                    vals = plsc.load_gather(rows_v, [t16, off + iota16])
                    vbuf[pl.ds(t * 16, 16)] = vals

                pltpu.sync_copy(
                    vbuf, ov_hbm.at[pl.ds((o + go) * E, _ICH * E)])

    return k(v128, w_flat, idx_t)


def _tc_body(xg_ref, wg_ref, wc_ref, w2_ref, w3_ref, c1_ref, c2_ref, c3_ref,
             o_ref):
    x = xg_ref[...]
    xb = x.astype(jnp.bfloat16)
    acc = lax.dot_general(xb, wc_ref[...], (((1,), (0,)), ((), ())),
                          preferred_element_type=jnp.float32)
    h1 = jnp.maximum(acc[:, :256] + c1_ref[...], 0.0)
    s = acc[:, 256:272]
    sumsq = jnp.sum(x * x, axis=1, keepdims=True)
    fm = 0.5 * (jnp.sum(s * s, axis=1, keepdims=True) - sumsq)
    wsum = jnp.sum(wg_ref[...], axis=1, keepdims=True)
    h2 = jnp.maximum(
        lax.dot_general(h1.astype(jnp.bfloat16), w2_ref[...],
                        (((1,), (0,)), ((), ())),
                        preferred_element_type=jnp.float32) + c2_ref[...], 0.0)
    h3 = jnp.sum(h2 * w3_ref[...], axis=1, keepdims=True)
    o_ref[...] = jax.nn.sigmoid(fm + wsum + h3 + c3_ref[...])


def _tc_call(xg, wgr, wc, w2, w3, c1, c2, c3, interpret=False):
    const = lambda i: (0, 0)
    return pl.pallas_call(
        _tc_body,
        grid=(B // _BB,),
        in_specs=[
            pl.BlockSpec((_BB, D0), lambda i: (i, 0)),
            pl.BlockSpec((_BB, F), lambda i: (i, 0)),
            pl.BlockSpec((D0, 272), const),
            pl.BlockSpec((256, 128), const),
            pl.BlockSpec((1, 128), const),
            pl.BlockSpec((1, 256), const),
            pl.BlockSpec((1, 128), const),
            pl.BlockSpec((1, 1), const),
        ],
        out_specs=pl.BlockSpec((_BB, 1), lambda i: (i, 0)),
        out_shape=jax.ShapeDtypeStruct((B, 1), jnp.float32),
        interpret=interpret,
    )(xg, wgr, wc, w2, w3, c1, c2, c3)


def _fold_weights(W1, b1, W2, b2, W3, b3, w0,
                  bn1a_g, bn1a_b, bn1a_m, bn1a_v, bn1b_g, bn1b_b, bn1b_m,
                  bn1b_v, bn2a_g, bn2a_b, bn2a_m, bn2a_v, bn2b_g, bn2b_b,
                  bn2b_m, bn2b_v):
    def affine(g_a, b_a, m_a, v_a, g_b, b_b, m_b, v_b):
        sa = g_a * lax.rsqrt(v_a + 1e-5)
        ta = b_a - m_a * sa
        sb = g_b * lax.rsqrt(v_b + 1e-5)
        tb = b_b - m_b * sb
        return sa * sb, ta * sb + tb

    s1, t1 = affine(bn1a_g, bn1a_b, bn1a_m, bn1a_v,
                    bn1b_g, bn1b_b, bn1b_m, bn1b_v)
    s2, t2 = affine(bn2a_g, bn2a_b, bn2a_m, bn2a_v,
                    bn2b_g, bn2b_b, bn2b_m, bn2b_v)
    w1f = (W1 * s1[:, None]).T
    ident = jnp.tile(jnp.eye(E, dtype=jnp.float32), (F, 1))
    wc = jnp.concatenate([w1f, ident], axis=1).astype(jnp.bfloat16)
    c1 = (b1 * s1 + t1)[None, :]
    w2f = ((W2 * s2[:, None]).T).astype(jnp.bfloat16)
    c2 = (b2 * s2 + t2)[None, :]
    c3 = (b3 + w0).reshape(1, 1)
    return wc, w2f, W3, c1, c2, c3


def kernel(inputs, w_table, v_table, w0, W1, b1, W2, b2, W3, b3,
           bn1a_g, bn1a_b, bn1a_m, bn1a_v, bn1b_g, bn1b_b, bn1b_m, bn1b_v,
           bn2a_g, bn2a_b, bn2a_m, bn2a_v, bn2b_g, bn2b_b, bn2b_m, bn2b_v):
    v128 = lax.optimization_barrier(v_table.reshape(VROWS, 128))
    vgf, wgf = _sc_gather(v128, w_table.reshape(-1), inputs)
    xg = vgf.reshape(B, D0)
    wgr = wgf.reshape(B, F)
    wc, w2f, w3, c1, c2, c3 = _fold_weights(
        W1, b1, W2, b2, W3, b3, w0,
        bn1a_g, bn1a_b, bn1a_m, bn1a_v, bn1b_g, bn1b_b, bn1b_m, bn1b_v,
        bn2a_g, bn2a_b, bn2a_m, bn2a_v, bn2b_g, bn2b_b, bn2b_m, bn2b_v)
    return _tc_call(xg, wgr, wc, w2f, w3, c1, c2, c3)
